# baseline (speedup 1.0000x reference)
import functools

import jax
import jax.numpy as jnp
from jax import lax
from jax.experimental import pallas as pl
from jax.experimental.pallas import tpu as pltpu
from jax.experimental.pallas import tpu_sc as plsc

LANES = 16
CHUNK = 8


def _build_sc_call(B, L, E):
    RIN = L * E
    ROUT = (2 * L + 1) * E
    info = plsc.get_sparse_core_info()
    NC, NS = info.num_cores, info.num_subcores
    NW = NC * NS
    NCHUNK = RIN // CHUNK
    assert RIN % CHUNK == 0 and NCHUNK % NW == 0 and E % LANES == 0
    CPW = NCHUNK // NW
    NVC = B // LANES

    def body(x1_hbm, x2_hbm, pat_hbm, out_hbm,
             ia, ib, oa, ob, pat,
             sina, sinb, souta, soutb):
        wid = lax.axis_index("s") * NC + lax.axis_index("c")
        ibuf = (ia, ib)
        obuf = (oa, ob)
        sin = (sina, sinb)
        sout = (souta, soutb)
        srcs = (x1_hbm, x2_hbm)
        outoff = (E, E + RIN)
        patbase = (0, E * LANES)

        pltpu.sync_copy(pat_hbm, pat)

        def rs_of(c):
            return (wid * CPW + c) * CHUNK

        def issue_in(c, p):
            pltpu.async_copy(srcs[p].at[pl.ds(rs_of(c), CHUNK)],
                             ibuf[p], sin[p])

        def wait_in(p):
            pltpu.make_async_copy(srcs[p].at[pl.ds(0, CHUNK)],
                                  ibuf[p], sin[p]).wait()

        def issue_out(c, p):
            pltpu.async_copy(obuf[p],
                             out_hbm.at[pl.ds(outoff[p] + rs_of(c), CHUNK)],
                             sout[p])

        def wait_out(p):
            pltpu.make_async_copy(obuf[p],
                                  out_hbm.at[pl.ds(0, CHUNK)],
                                  sout[p]).wait()

        def compute(c, p):
            r, w = ibuf[p], obuf[p]
            pb = patbase[p] + (rs_of(c) & (E - 1)) * LANES
            splats = [pat[pl.ds(pb + j * LANES, LANES)] for j in range(CHUNK)]

            @plsc.parallel_loop(0, NVC, step=1, unroll=1)
            def _(v):
                s = pl.ds(v * LANES, LANES)
                for j in range(CHUNK):
                    w[j, s] = r[j, s] + splats[j]

        issue_in(0, 0)
        issue_in(0, 1)

        def loop_body(c, carry):
            for p in range(2):
                wait_in(p)

                @pl.when(c > 0)
                def _():
                    wait_out(p)

                compute(c, p)
                issue_out(c, p)

                @pl.when(c < CPW - 1)
                def _():
                    issue_in(c + 1, p)
            return carry

        lax.fori_loop(0, CPW, loop_body, 0)
        wait_out(0)
        wait_out(1)

        @pl.when(wid < E // CHUNK)
        def _():
            base = wid * CHUNK

            @plsc.parallel_loop(0, NVC, step=1, unroll=1)
            def _(v):
                s = pl.ds(v * LANES, LANES)
                for j in range(CHUNK):
                    oa[j, s] = pat[pl.ds((2 * E + base + j) * LANES, LANES)]

            pltpu.sync_copy(oa, out_hbm.at[pl.ds(base, CHUNK)])

    mesh = plsc.VectorSubcoreMesh(core_axis_name="c", subcore_axis_name="s")
    return pl.kernel(
        body,
        mesh=mesh,
        out_type=jax.ShapeDtypeStruct((ROUT, B), jnp.float32),
        scratch_types=[
            pltpu.VMEM((CHUNK, B), jnp.float32),
            pltpu.VMEM((CHUNK, B), jnp.float32),
            pltpu.VMEM((CHUNK, B), jnp.float32),
            pltpu.VMEM((CHUNK, B), jnp.float32),
            pltpu.VMEM((3 * E * LANES,), jnp.float32),
            pltpu.SemaphoreType.DMA,
            pltpu.SemaphoreType.DMA,
            pltpu.SemaphoreType.DMA,
            pltpu.SemaphoreType.DMA,
        ],
    )


def kernel(x1, x2, cls_token, type_embeddings):
    B, L, E = x1.shape
    call = _build_sc_call(B, L, E)
    x1v = x1.transpose(1, 2, 0).reshape(L * E, B)
    x2v = x2.transpose(1, 2, 0).reshape(L * E, B)
    scal = jnp.concatenate(
        [type_embeddings.reshape(2 * E), cls_token.reshape(E)])
    pat = jnp.repeat(scal, LANES)
    outv = call(x1v, x2v, pat)
    return outv.reshape(2 * L + 1, E, B).transpose(2, 0, 1)


# speedup vs baseline: 1.1633x; 1.1633x over previous
import functools

import jax
import jax.numpy as jnp
from jax import lax
from jax.experimental import pallas as pl
from jax.experimental.pallas import tpu as pltpu
from jax.experimental.pallas import tpu_sc as plsc

LANES = 16
CHUNK = 16


def _build_sc_call(B, L, E):
    RIN = L * E
    ROUT = (2 * L + 1) * E
    info = plsc.get_sparse_core_info()
    NC, NS = info.num_cores, info.num_subcores
    NW = NC * NS
    NCHUNK = RIN // CHUNK
    assert RIN % CHUNK == 0 and NCHUNK % NW == 0 and E % LANES == 0
    CPW = NCHUNK // NW
    NVC = B // LANES

    def body(x1_hbm, x2_hbm, pat_hbm, out_hbm,
             ia, ib, oa, ob, pat,
             sina, sinb, souta, soutb):
        wid = lax.axis_index("s") * NC + lax.axis_index("c")
        ibuf = (ia, ib)
        obuf = (oa, ob)
        sin = (sina, sinb)
        sout = (souta, soutb)
        srcs = (x1_hbm, x2_hbm)
        outoff = (E, E + RIN)
        patbase = (0, E * LANES)

        pltpu.sync_copy(pat_hbm, pat)

        def rs_of(c):
            return (wid * CPW + c) * CHUNK

        def issue_in(c, p):
            pltpu.async_copy(srcs[p].at[pl.ds(rs_of(c), CHUNK)],
                             ibuf[p], sin[p])

        def wait_in(p):
            pltpu.make_async_copy(srcs[p].at[pl.ds(0, CHUNK)],
                                  ibuf[p], sin[p]).wait()

        def issue_out(c, p):
            pltpu.async_copy(obuf[p],
                             out_hbm.at[pl.ds(outoff[p] + rs_of(c), CHUNK)],
                             sout[p])

        def wait_out(p):
            pltpu.make_async_copy(obuf[p],
                                  out_hbm.at[pl.ds(0, CHUNK)],
                                  sout[p]).wait()

        def compute(c, p):
            r, w = ibuf[p], obuf[p]
            pb = patbase[p] + (rs_of(c) & (E - 1)) * LANES
            splats = [pat[pl.ds(pb + j * LANES, LANES)] for j in range(CHUNK)]

            @plsc.parallel_loop(0, NVC, step=1, unroll=1)
            def _(v):
                s = pl.ds(v * LANES, LANES)
                for j in range(CHUNK):
                    w[j, s] = r[j, s] + splats[j]

        issue_in(0, 0)
        issue_in(0, 1)

        def loop_body(c, carry):
            for p in range(2):
                wait_in(p)

                @pl.when(c > 0)
                def _():
                    wait_out(p)

                compute(c, p)
                issue_out(c, p)

                @pl.when(c < CPW - 1)
                def _():
                    issue_in(c + 1, p)
            return carry

        lax.fori_loop(0, CPW, loop_body, 0)
        wait_out(0)
        wait_out(1)

        @pl.when(wid < E // CHUNK)
        def _():
            base = wid * CHUNK

            @plsc.parallel_loop(0, NVC, step=1, unroll=1)
            def _(v):
                s = pl.ds(v * LANES, LANES)
                for j in range(CHUNK):
                    oa[j, s] = pat[pl.ds((2 * E + base + j) * LANES, LANES)]

            pltpu.sync_copy(oa, out_hbm.at[pl.ds(base, CHUNK)])

    mesh = plsc.VectorSubcoreMesh(core_axis_name="c", subcore_axis_name="s")
    return pl.kernel(
        body,
        mesh=mesh,
        out_type=jax.ShapeDtypeStruct((ROUT, B), jnp.float32),
        scratch_types=[
            pltpu.VMEM((CHUNK, B), jnp.float32),
            pltpu.VMEM((CHUNK, B), jnp.float32),
            pltpu.VMEM((CHUNK, B), jnp.float32),
            pltpu.VMEM((CHUNK, B), jnp.float32),
            pltpu.VMEM((3 * E * LANES,), jnp.float32),
            pltpu.SemaphoreType.DMA,
            pltpu.SemaphoreType.DMA,
            pltpu.SemaphoreType.DMA,
            pltpu.SemaphoreType.DMA,
        ],
    )


def kernel(x1, x2, cls_token, type_embeddings):
    B, L, E = x1.shape
    call = _build_sc_call(B, L, E)
    x1v = x1.transpose(1, 2, 0).reshape(L * E, B)
    x2v = x2.transpose(1, 2, 0).reshape(L * E, B)
    scal = jnp.concatenate(
        [type_embeddings.reshape(2 * E), cls_token.reshape(E)])
    pat = jnp.repeat(scal, LANES)
    outv = call(x1v, x2v, pat)
    return outv.reshape(2 * L + 1, E, B).transpose(2, 0, 1)
